# Initial kernel scaffold; baseline (speedup 1.0000x reference)
import jax
import jax.numpy as jnp
from jax.experimental import pallas as pl


def kernel(h, edge_index, largest, W_ft, b_ft, W1, b1, W2, b2, W3, b3):
    raise NotImplementedError("write your pallas kernel here")


# speedup vs baseline: 2.7165x; 2.7165x over previous
import functools

import jax
import jax.numpy as jnp
from jax.experimental import pallas as pl
from jax.experimental.pallas import tpu as pltpu
from jax.experimental.pallas import tpu_sc as plsc

N_NODES = 10000
N_EDGES = 320000
K_NUM = 160000
NC = 2
NS = 16

BF = jnp.bfloat16

DEG_CH = 5000
DEG_CHUNKS = 8


def _deg_body(ei_ref, zeros_ref, ones_ref, out_ref, idx_v, ones_v, hist_sh):
    c = jax.lax.axis_index("c")
    s = jax.lax.axis_index("s")

    @pl.when(c == 0)
    def _():
        @pl.when(s == 0)
        def _():
            pltpu.sync_copy(zeros_ref, hist_sh)

        pltpu.sync_copy(ones_ref, ones_v)
        plsc.subcore_barrier()
        for j in range(DEG_CHUNKS):
            base = s * (DEG_CH * DEG_CHUNKS) + j * DEG_CH
            pltpu.sync_copy(ei_ref.at[pl.ds(base, DEG_CH)], idx_v)
            pltpu.sync_copy(ones_v, hist_sh.at[idx_v], add=True)
        plsc.subcore_barrier()

        @pl.when(s == 0)
        def _():
            pltpu.sync_copy(hist_sh, out_ref)


def _deg_counts(ei_flat):
    mesh = plsc.VectorSubcoreMesh(core_axis_name="c", subcore_axis_name="s")
    kern = functools.partial(
        pl.kernel,
        mesh=mesh,
        out_type=jax.ShapeDtypeStruct((N_NODES,), jnp.float32),
        scratch_types=[
            pltpu.VMEM((DEG_CH,), jnp.int32),
            pltpu.VMEM((DEG_CH,), jnp.float32),
            pltpu.VMEM_SHARED((N_NODES,), jnp.float32),
        ],
    )(_deg_body)
    zeros = jnp.zeros((N_NODES,), jnp.float32)
    ones = jnp.ones((DEG_CH,), jnp.float32)
    return kern(ei_flat, zeros, ones)


def _feat_body(deg_ref, h_ref, wft_ref, bft_ref, out_ref):
    x = jnp.concatenate([deg_ref[...], h_ref[...]], axis=1)
    acc = jax.lax.dot_general(x.astype(BF), wft_ref[...].astype(BF),
                              (((1,), (0,)), ((), ())),
                              preferred_element_type=jnp.float32)
    feat = acc + bft_ref[...][None, :]
    out_ref[...] = jnp.concatenate(
        [feat, jnp.zeros((feat.shape[0], 64), jnp.float32)], axis=1)


def _node_feat(deg_col, h, W_ft, b_ft):
    return pl.pallas_call(
        _feat_body,
        out_shape=jax.ShapeDtypeStruct((N_NODES, 128), jnp.float32),
    )(deg_col, h, W_ft, b_ft)


G_CH = 1000
G_CHUNKS = 20


def _gather_body(feat_ref, idx_ref, out_ref, idx_v, rows_v, sem):
    c = jax.lax.axis_index("c")
    s = jax.lax.axis_index("s")
    wid = s * NC + c
    for j in range(G_CHUNKS):
        base = wid * (G_CH * G_CHUNKS) + j * G_CH
        pltpu.sync_copy(idx_ref.at[pl.ds(base, G_CH)], idx_v)
        pltpu.async_copy(feat_ref.at[idx_v], rows_v, sem).wait()
        pltpu.sync_copy(rows_v, out_ref.at[pl.ds(base, G_CH)])


def _gather_rows(feat, ei_flat):
    mesh = plsc.VectorSubcoreMesh(core_axis_name="c", subcore_axis_name="s")
    kern = functools.partial(
        pl.kernel,
        mesh=mesh,
        out_type=jax.ShapeDtypeStruct((2 * N_EDGES, 128), jnp.float32),
        scratch_types=[
            pltpu.VMEM((G_CH,), jnp.int32),
            pltpu.VMEM((G_CH, 128), jnp.float32),
            pltpu.SemaphoreType.DMA,
        ],
    )(_gather_body)
    return kern(feat, ei_flat)


SBLK = 512


def _rowsum64(p):
    acc = p[:, 0:8]
    for s in range(1, 8):
        acc = acc + p[:, 8 * s:8 * s + 8]
    acc = acc[:, 0:4] + acc[:, 4:8]
    acc = acc[:, 0:2] + acc[:, 2:4]
    return acc[:, 0:1] + acc[:, 1:2]


def _score_body(rf_ref, cf_ref, w1_ref, b1_ref, w2_ref, b2_ref, w3_ref,
                b3_ref, s_o):
    rf = rf_ref[...][:, :64]
    cf = cf_ref[...][:, :64]
    dot = _rowsum64(rf * cf)
    na = jnp.sqrt(_rowsum64(rf * rf))
    nb = jnp.sqrt(_rowsum64(cf * cf))
    sim = dot / (jnp.maximum(na, 1e-8) * jnp.maximum(nb, 1e-8))
    link = jnp.concatenate([sim, rf, cf], axis=1)
    mm = lambda a, w: jax.lax.dot_general(a.astype(BF), w.astype(BF),
                                          (((1,), (0,)), ((), ())),
                                          preferred_element_type=jnp.float32)
    l1 = jax.nn.relu(mm(link, w1_ref[...]) + b1_ref[...][None, :])
    l2 = jax.nn.relu(mm(l1, w2_ref[...]) + b2_ref[...][None, :])
    logit = mm(l2, w3_ref[...]) + b3_ref[...][None, :]
    s_o[...] = jax.nn.sigmoid(logit)


def _edge_scores(gathered, W1, b1, W2, b2, W3, b3):
    return pl.pallas_call(
        _score_body,
        out_shape=jax.ShapeDtypeStruct((N_EDGES, 1), jnp.float32),
        grid=(N_EDGES // SBLK,),
        in_specs=[pl.BlockSpec((SBLK, 128), lambda i: (i, 0)),
                  pl.BlockSpec((SBLK, 128), lambda i: (i + N_EDGES // SBLK, 0)),
                  pl.BlockSpec((129, 64), lambda i: (0, 0)),
                  pl.BlockSpec((64,), lambda i: (0,)),
                  pl.BlockSpec((64, 32), lambda i: (0, 0)),
                  pl.BlockSpec((32,), lambda i: (0,)),
                  pl.BlockSpec((32, 1), lambda i: (0, 0)),
                  pl.BlockSpec((1,), lambda i: (0,))],
        out_specs=pl.BlockSpec((SBLK, 1), lambda i: (i, 0)),
    )(gathered, gathered, W1, b1, W2, b2, W3, b3)


def kernel(h, edge_index, largest, W_ft, b_ft, W1, b1, W2, b2, W3, b3):
    ei_flat = edge_index.reshape(-1)
    deg_col = _deg_counts(ei_flat).reshape(N_NODES, 1)
    feat = _node_feat(deg_col, h, W_ft, b_ft)
    gathered = _gather_rows(feat, ei_flat)
    scores = _edge_scores(gathered, W1, b1, W2, b2, W3, b3)[:, 0]

    sign = jnp.where(largest, jnp.float32(1.0), jnp.float32(-1.0))
    top_values, idxs = jax.lax.top_k(scores * sign, K_NUM)
    values = top_values * sign
    sel_edge_index = edge_index[:, idxs]
    edge_mask = values[:, None]
    return (sel_edge_index, h, scores, edge_mask)
